# baseline (speedup 1.0000x reference)
import functools

import jax
import jax.numpy as jnp
from jax import lax
from jax.experimental import pallas as pl
from jax.experimental.pallas import tpu as pltpu
from jax.experimental.pallas import tpu_sc as plsc

_N = 4096
_D = 96
_M = 16384
_QB = 128
_C = 2048
_G = 32
_INF = 3.0e38
_BIGI = 2**30

_NCORES = 2
_NSUB = 16
_NW = _NCORES * _NSUB
_QPW = _N // _NW


def _idx_body(q_ref, mb_ref, idx_ref, m2_ref):
    q = q_ref[...]
    nc = _M // _C
    ng = _C // _G

    @pl.when(pl.program_id(0) == 0)
    def _build_m2():
        for c in range(nc):
            mb = mb_ref[pl.ds(c * _C, _C), :]
            m2_ref[pl.ds(c * _C, _C), :] = jnp.sum(mb * mb, axis=1,
                                                   keepdims=True)

    gvals = []
    gidxs = []
    for c in range(nc):
        mb = mb_ref[pl.ds(c * _C, _C), :]
        qm = lax.dot_general(mb, q, (((1,), (1,)), ((), ())),
                             preferred_element_type=jnp.float32)
        s = m2_ref[pl.ds(c * _C, _C), :] - 2.0 * qm
        s3 = s.reshape(ng, _G, _QB)
        iota1 = lax.broadcasted_iota(jnp.int32, (ng, _G, _QB), 1)
        g1v = jnp.min(s3, axis=1, keepdims=True)
        m1 = s3 <= g1v
        g1i = jnp.min(jnp.where(m1, iota1, _BIGI), axis=1)
        s3b = jnp.where(m1, _INF, s3)
        g2v = jnp.min(s3b, axis=1, keepdims=True)
        g2i = jnp.min(jnp.where(s3b <= g2v, iota1, _BIGI), axis=1)
        base = lax.broadcasted_iota(jnp.int32, (ng, _QB), 0) * _G + c * _C
        gvals.append(jnp.concatenate([g1v[:, 0, :], g2v[:, 0, :]], axis=0))
        gidxs.append(jnp.concatenate([base + g1i, base + g2i], axis=0))
    cv = jnp.concatenate(gvals, axis=0)
    ci = jnp.concatenate(gidxs, axis=0)
    ncand = 2 * ng * nc
    iota0 = lax.broadcasted_iota(jnp.int32, (ncand, _QB), 0)
    rows = []
    for _ in range(4):
        m = jnp.min(cv, axis=0, keepdims=True)
        pos = jnp.min(jnp.where(cv <= m, iota0, _BIGI), axis=0,
                      keepdims=True)
        hit = iota0 == pos
        rows.append(jnp.min(jnp.where(hit, ci, _BIGI), axis=0, keepdims=True))
        cv = jnp.where(hit, _INF, cv)
    zero = jnp.zeros((4, _QB), jnp.int32)
    idx_ref[...] = jnp.concatenate(rows + [zero], axis=0)


@functools.cache
def _make_gather_mean():
    @functools.partial(
        pl.kernel,
        mesh=plsc.VectorSubcoreMesh(core_axis_name="c", subcore_axis_name="s"),
        out_type=jax.ShapeDtypeStruct((_N, _D), jnp.float32),
        scratch_types=[
            pltpu.VMEM((4, _QPW), jnp.int32),
            pltpu.VMEM((_QPW, 128), jnp.float32),
            pltpu.VMEM((_QPW, 128), jnp.float32),
            pltpu.VMEM((_QPW, 128), jnp.float32),
            pltpu.VMEM((_QPW, 128), jnp.float32),
            pltpu.VMEM((_QPW, _D), jnp.float32),
            pltpu.SemaphoreType.DMA,
        ],
    )
    def _gather_mean(idx_hbm, mb_hbm, out_hbm, idx_v,
                     r0, r1, r2, r3, out_v, sem):
        wid = lax.axis_index("s") * _NCORES + lax.axis_index("c")
        base = wid * _QPW
        pltpu.sync_copy(idx_hbm.at[pl.ds(0, 4), pl.ds(base, _QPW)], idx_v)
        rows = (r0, r1, r2, r3)
        copies = [pltpu.async_copy(mb_hbm.at[idx_v.at[k]], rows[k], sem)
                  for k in range(4)]
        for cp in copies:
            cp.wait()

        def body(qi, carry):
            for dk in range(_D // 16):
                sl = pl.ds(dk * 16, 16)
                out_v[qi, sl] = (r0[qi, sl] + r1[qi, sl]
                                 + r2[qi, sl] + r3[qi, sl]) * 0.25
            return carry

        lax.fori_loop(0, _QPW, body, 0)
        pltpu.sync_copy(out_v, out_hbm.at[pl.ds(base, _QPW)])

    return _gather_mean


def _t9_body(q_ref, n_ref, wq_ref, wn_ref, t_ref):
    t_ref[...] = (
        lax.dot_general(q_ref[...], wq_ref[...], (((1,), (0,)), ((), ())),
                        preferred_element_type=jnp.float32)
        + lax.dot_general(n_ref[...], wn_ref[...], (((1,), (0,)), ((), ())),
                          preferred_element_type=jnp.float32))


def _shift_body(t_ref, b_ref, out_ref):
    zrow = jnp.zeros((1, 64), jnp.float32)
    zcol = jnp.zeros((64, 1), jnp.float32)
    acc = jnp.zeros((64, 64), jnp.float32)
    for ky in range(3):
        for kx in range(3):
            p = t_ref[ky * 3 + kx]
            if ky == 0:
                p = jnp.concatenate([zrow, p[:63, :]], axis=0)
            elif ky == 2:
                p = jnp.concatenate([p[1:, :], zrow], axis=0)
            if kx == 0:
                p = jnp.concatenate([zcol, p[:, :63]], axis=1)
            elif kx == 2:
                p = jnp.concatenate([p[:, 1:], zcol], axis=1)
            acc = acc + p
    out_ref[...] = acc + b_ref[0, 0]


def kernel(query, memory_bank, fcn_w, fcn_b):
    idx_t = pl.pallas_call(
        _idx_body,
        grid=(_N // _QB,),
        in_specs=[
            pl.BlockSpec((_QB, _D), lambda i: (i, 0)),
            pl.BlockSpec((_M, _D), lambda i: (0, 0)),
        ],
        out_specs=pl.BlockSpec((8, _QB), lambda i: (0, i)),
        out_shape=jax.ShapeDtypeStruct((8, _N), jnp.int32),
        scratch_shapes=[pltpu.VMEM((_M, 1), jnp.float32)],
        compiler_params=pltpu.CompilerParams(
            dimension_semantics=("arbitrary",)),
    )(query, memory_bank)

    mb_pad = jnp.pad(memory_bank, ((0, 0), (0, 128 - _D)))
    nmean = _make_gather_mean()(idx_t, mb_pad)

    w = fcn_w[0].reshape(2 * _D, 9)
    wq, wn = w[:_D], w[_D:]
    t9 = pl.pallas_call(
        _t9_body,
        grid=(_N // _QB,),
        in_specs=[
            pl.BlockSpec((_QB, _D), lambda i: (i, 0)),
            pl.BlockSpec((_QB, _D), lambda i: (i, 0)),
            pl.BlockSpec((_D, 9), lambda i: (0, 0)),
            pl.BlockSpec((_D, 9), lambda i: (0, 0)),
        ],
        out_specs=pl.BlockSpec((_QB, 9), lambda i: (i, 0)),
        out_shape=jax.ShapeDtypeStruct((_N, 9), jnp.float32),
    )(query, nmean, wq, wn)

    t3 = t9.T.reshape(9, 64, 64)
    out = pl.pallas_call(
        _shift_body,
        in_specs=[
            pl.BlockSpec((9, 64, 64), lambda: (0, 0, 0)),
            pl.BlockSpec((1, 1), lambda: (0, 0), memory_space=pltpu.SMEM),
        ],
        out_specs=pl.BlockSpec((64, 64), lambda: (0, 0)),
        out_shape=jax.ShapeDtypeStruct((64, 64), jnp.float32),
    )(t3, fcn_b.reshape(1, 1))
    return out.reshape(1, 1, 64, 64)


# speedup vs baseline: 1.2167x; 1.2167x over previous
import functools

import jax
import jax.numpy as jnp
from jax import lax
from jax.experimental import pallas as pl
from jax.experimental.pallas import tpu as pltpu
from jax.experimental.pallas import tpu_sc as plsc

_N = 4096
_D = 96
_M = 16384
_QB = 256
_C = 2048
_G = 32
_INF = 3.0e38
_BIGI = 2**30
_MAXI = 2**31 - 1

_NCORES = 2
_NSUB = 16
_NW = _NCORES * _NSUB
_QPW = _N // _NW


def _idx_body(q_ref, mb_ref, idx_ref, m2_ref):
    q = q_ref[...]
    nc = _M // _C
    ng = _C // _G

    @pl.when(pl.program_id(0) == 0)
    def _build_m2():
        for c in range(nc):
            mb = mb_ref[pl.ds(c * _C, _C), :]
            m2_ref[pl.ds(c * _C, _C), :] = jnp.sum(mb * mb, axis=1,
                                                   keepdims=True)

    gkeys = []
    for c in range(nc):
        mb = mb_ref[pl.ds(c * _C, _C), :]
        qm = lax.dot_general(mb, q, (((1,), (1,)), ((), ())),
                             preferred_element_type=jnp.float32)
        s = m2_ref[pl.ds(c * _C, _C), :] - 2.0 * qm
        b = lax.bitcast_convert_type(s, jnp.int32)
        k = b ^ ((b >> 31) & jnp.int32(0x7FFFFFFF))
        k3 = k.reshape(ng, _G, _QB)
        lane = lax.broadcasted_iota(jnp.int32, (ng, _G, _QB), 1)
        k3 = (k3 & jnp.int32(~31)) | lane
        g1 = jnp.min(k3, axis=1, keepdims=True)
        k3b = jnp.where(k3 <= g1, _MAXI, k3)
        g2 = jnp.min(k3b, axis=1)
        gkeys.append(jnp.concatenate([g1[:, 0, :], g2], axis=0))
    ck = jnp.concatenate(gkeys, axis=0)
    ncand = 2 * ng * nc
    iota0 = lax.broadcasted_iota(jnp.int32, (ncand, _QB), 0)
    rows = []
    for _ in range(4):
        m = jnp.min(ck, axis=0, keepdims=True)
        hit = ck == m
        pos = jnp.min(jnp.where(hit, iota0, _BIGI), axis=0, keepdims=True)
        grp = (pos // (2 * ng)) * ng + pos % ng
        rows.append(grp * _G + (m & 31))
        ck = jnp.where(iota0 == pos, _MAXI, ck)
    zero = jnp.zeros((4, _QB), jnp.int32)
    idx_ref[...] = jnp.concatenate(rows + [zero], axis=0)


@functools.cache
def _make_gather_mean():
    @functools.partial(
        pl.kernel,
        mesh=plsc.VectorSubcoreMesh(core_axis_name="c", subcore_axis_name="s"),
        out_type=jax.ShapeDtypeStruct((_N, _D), jnp.float32),
        scratch_types=[
            pltpu.VMEM((4, _QPW), jnp.int32),
            pltpu.VMEM((_QPW, 128), jnp.float32),
            pltpu.VMEM((_QPW, 128), jnp.float32),
            pltpu.VMEM((_QPW, 128), jnp.float32),
            pltpu.VMEM((_QPW, 128), jnp.float32),
            pltpu.VMEM((_QPW, _D), jnp.float32),
            pltpu.SemaphoreType.DMA,
        ],
    )
    def _gather_mean(idx_hbm, mb_hbm, out_hbm, idx_v,
                     r0, r1, r2, r3, out_v, sem):
        wid = lax.axis_index("s") * _NCORES + lax.axis_index("c")
        base = wid * _QPW
        pltpu.sync_copy(idx_hbm.at[pl.ds(0, 4), pl.ds(base, _QPW)], idx_v)
        rows = (r0, r1, r2, r3)
        copies = [pltpu.async_copy(mb_hbm.at[idx_v.at[k]], rows[k], sem)
                  for k in range(4)]
        for cp in copies:
            cp.wait()

        def body(qi, carry):
            for dk in range(_D // 16):
                sl = pl.ds(dk * 16, 16)
                out_v[qi, sl] = (r0[qi, sl] + r1[qi, sl]
                                 + r2[qi, sl] + r3[qi, sl]) * 0.25
            return carry

        lax.fori_loop(0, _QPW, body, 0)
        pltpu.sync_copy(out_v, out_hbm.at[pl.ds(base, _QPW)])

    return _gather_mean


def _t9_body(q_ref, n_ref, wq_ref, wn_ref, t_ref):
    t_ref[...] = (
        lax.dot_general(q_ref[...], wq_ref[...], (((1,), (0,)), ((), ())),
                        preferred_element_type=jnp.float32)
        + lax.dot_general(n_ref[...], wn_ref[...], (((1,), (0,)), ((), ())),
                          preferred_element_type=jnp.float32))


def _shift_body(t_ref, b_ref, out_ref):
    zrow = jnp.zeros((1, 64), jnp.float32)
    zcol = jnp.zeros((64, 1), jnp.float32)
    acc = jnp.zeros((64, 64), jnp.float32)
    for ky in range(3):
        for kx in range(3):
            p = t_ref[ky * 3 + kx]
            if ky == 0:
                p = jnp.concatenate([zrow, p[:63, :]], axis=0)
            elif ky == 2:
                p = jnp.concatenate([p[1:, :], zrow], axis=0)
            if kx == 0:
                p = jnp.concatenate([zcol, p[:, :63]], axis=1)
            elif kx == 2:
                p = jnp.concatenate([p[:, 1:], zcol], axis=1)
            acc = acc + p
    out_ref[...] = acc + b_ref[0, 0]


def kernel(query, memory_bank, fcn_w, fcn_b):
    idx_t = pl.pallas_call(
        _idx_body,
        grid=(_N // _QB,),
        in_specs=[
            pl.BlockSpec((_QB, _D), lambda i: (i, 0)),
            pl.BlockSpec((_M, _D), lambda i: (0, 0)),
        ],
        out_specs=pl.BlockSpec((8, _QB), lambda i: (0, i)),
        out_shape=jax.ShapeDtypeStruct((8, _N), jnp.int32),
        scratch_shapes=[pltpu.VMEM((_M, 1), jnp.float32)],
        compiler_params=pltpu.CompilerParams(
            dimension_semantics=("arbitrary",)),
    )(query, memory_bank)

    mb_pad = jnp.pad(memory_bank, ((0, 0), (0, 128 - _D)))
    nmean = _make_gather_mean()(idx_t, mb_pad)

    w = fcn_w[0].reshape(2 * _D, 9)
    wq, wn = w[:_D], w[_D:]
    t9 = pl.pallas_call(
        _t9_body,
        grid=(_N // _QB,),
        in_specs=[
            pl.BlockSpec((_QB, _D), lambda i: (i, 0)),
            pl.BlockSpec((_QB, _D), lambda i: (i, 0)),
            pl.BlockSpec((_D, 9), lambda i: (0, 0)),
            pl.BlockSpec((_D, 9), lambda i: (0, 0)),
        ],
        out_specs=pl.BlockSpec((_QB, 9), lambda i: (i, 0)),
        out_shape=jax.ShapeDtypeStruct((_N, 9), jnp.float32),
    )(query, nmean, wq, wn)

    t3 = t9.T.reshape(9, 64, 64)
    out = pl.pallas_call(
        _shift_body,
        in_specs=[
            pl.BlockSpec((9, 64, 64), lambda: (0, 0, 0)),
            pl.BlockSpec((1, 1), lambda: (0, 0), memory_space=pltpu.SMEM),
        ],
        out_specs=pl.BlockSpec((64, 64), lambda: (0, 0)),
        out_shape=jax.ShapeDtypeStruct((64, 64), jnp.float32),
    )(t3, fcn_b.reshape(1, 1))
    return out.reshape(1, 1, 64, 64)


# speedup vs baseline: 1.4525x; 1.1938x over previous
import functools

import jax
import jax.numpy as jnp
from jax import lax
from jax.experimental import pallas as pl
from jax.experimental.pallas import tpu as pltpu
from jax.experimental.pallas import tpu_sc as plsc

_N = 4096
_D = 96
_M = 16384
_QB = 256
_C = 2048
_G = 32
_INF = 3.0e38
_BIGI = 2**30
_MAXI = 2**31 - 1

_NCORES = 2
_NSUB = 16
_NW = _NCORES * _NSUB
_QPW = _N // _NW


def _idx_body(q_ref, mb_ref, idx_ref, m2_ref):
    q = q_ref[...]
    nc = _M // _C
    ng = _C // _G

    @pl.when(pl.program_id(0) == 0)
    def _build_m2():
        for c in range(nc):
            mb = mb_ref[pl.ds(c * _C, _C), :]
            m2_ref[pl.ds(c * _C, _C), :] = jnp.sum(mb * mb, axis=1,
                                                   keepdims=True)

    qn = -2.0 * q
    ng = _C // _G
    pos5 = lax.broadcasted_iota(jnp.int32, (_C, _QB), 0) // ng
    gkeys = []
    for c in range(nc):
        mb = mb_ref[pl.ds(c * _C, _C), :]
        qm = lax.dot_general(mb, qn, (((1,), (1,)), ((), ())),
                             preferred_element_type=jnp.float32)
        s = m2_ref[pl.ds(c * _C, _C), :] + qm
        b = lax.bitcast_convert_type(s, jnp.int32)
        k = b ^ ((b >> 31) & jnp.int32(0x7FFFFFFF))
        k = (k & jnp.int32(~31)) | pos5
        h = _C // 2
        a1 = jnp.minimum(k[:h], k[h:])
        a2 = jnp.maximum(k[:h], k[h:])
        while h > ng:
            h //= 2
            b1, c1 = a1[:h], a1[h:]
            n1 = jnp.minimum(b1, c1)
            a2 = jnp.minimum(jnp.maximum(b1, c1),
                             jnp.minimum(a2[:h], a2[h:]))
            a1 = n1
        gkeys.append(jnp.concatenate([a1, a2], axis=0))
    ck = jnp.concatenate(gkeys, axis=0)
    ncand = 2 * ng * nc
    iota0 = lax.broadcasted_iota(jnp.int32, (ncand, _QB), 0)
    rows = []
    for _ in range(4):
        m = jnp.min(ck, axis=0, keepdims=True)
        hit = ck == m
        pos = jnp.min(jnp.where(hit, iota0, _BIGI), axis=0, keepdims=True)
        row = (pos // (2 * ng)) * _C + pos % ng + ng * (m & 31)
        rows.append(row)
        ck = jnp.where(iota0 == pos, _MAXI, ck)
    zero = jnp.zeros((4, _QB), jnp.int32)
    idx_ref[...] = jnp.concatenate(rows + [zero], axis=0)


@functools.cache
def _make_gather_mean():
    @functools.partial(
        pl.kernel,
        mesh=plsc.VectorSubcoreMesh(core_axis_name="c", subcore_axis_name="s"),
        out_type=jax.ShapeDtypeStruct((_N, _D), jnp.float32),
        scratch_types=[
            pltpu.VMEM((4, _QPW), jnp.int32),
            pltpu.VMEM((_QPW, 128), jnp.float32),
            pltpu.VMEM((_QPW, 128), jnp.float32),
            pltpu.VMEM((_QPW, 128), jnp.float32),
            pltpu.VMEM((_QPW, 128), jnp.float32),
            pltpu.VMEM((_QPW, _D), jnp.float32),
            pltpu.SemaphoreType.DMA,
        ],
    )
    def _gather_mean(idx_hbm, mb_hbm, out_hbm, idx_v,
                     r0, r1, r2, r3, out_v, sem):
        wid = lax.axis_index("s") * _NCORES + lax.axis_index("c")
        base = wid * _QPW
        pltpu.sync_copy(idx_hbm.at[pl.ds(0, 4), pl.ds(base, _QPW)], idx_v)
        rows = (r0, r1, r2, r3)
        copies = [pltpu.async_copy(mb_hbm.at[idx_v.at[k]], rows[k], sem)
                  for k in range(4)]
        for cp in copies:
            cp.wait()

        def body(qi, carry):
            for dk in range(_D // 16):
                sl = pl.ds(dk * 16, 16)
                out_v[qi, sl] = (r0[qi, sl] + r1[qi, sl]
                                 + r2[qi, sl] + r3[qi, sl]) * 0.25
            return carry

        lax.fori_loop(0, _QPW, body, 0)
        pltpu.sync_copy(out_v, out_hbm.at[pl.ds(base, _QPW)])

    return _gather_mean


def _t9_body(q_ref, n_ref, wq_ref, wn_ref, t_ref):
    t_ref[...] = (
        lax.dot_general(q_ref[...], wq_ref[...], (((1,), (0,)), ((), ())),
                        preferred_element_type=jnp.float32)
        + lax.dot_general(n_ref[...], wn_ref[...], (((1,), (0,)), ((), ())),
                          preferred_element_type=jnp.float32))


def _shift_body(t_ref, b_ref, out_ref):
    zrow = jnp.zeros((1, 64), jnp.float32)
    zcol = jnp.zeros((64, 1), jnp.float32)
    acc = jnp.zeros((64, 64), jnp.float32)
    for ky in range(3):
        for kx in range(3):
            p = t_ref[ky * 3 + kx]
            if ky == 0:
                p = jnp.concatenate([zrow, p[:63, :]], axis=0)
            elif ky == 2:
                p = jnp.concatenate([p[1:, :], zrow], axis=0)
            if kx == 0:
                p = jnp.concatenate([zcol, p[:, :63]], axis=1)
            elif kx == 2:
                p = jnp.concatenate([p[:, 1:], zcol], axis=1)
            acc = acc + p
    out_ref[...] = acc + b_ref[0, 0]


def kernel(query, memory_bank, fcn_w, fcn_b):
    idx_t = pl.pallas_call(
        _idx_body,
        grid=(_N // _QB,),
        in_specs=[
            pl.BlockSpec((_QB, _D), lambda i: (i, 0)),
            pl.BlockSpec((_M, _D), lambda i: (0, 0)),
        ],
        out_specs=pl.BlockSpec((8, _QB), lambda i: (0, i)),
        out_shape=jax.ShapeDtypeStruct((8, _N), jnp.int32),
        scratch_shapes=[pltpu.VMEM((_M, 1), jnp.float32)],
        compiler_params=pltpu.CompilerParams(
            dimension_semantics=("arbitrary",)),
    )(query, memory_bank)

    mb_pad = jnp.pad(memory_bank, ((0, 0), (0, 128 - _D)))
    nmean = _make_gather_mean()(idx_t, mb_pad)

    w = fcn_w[0].reshape(2 * _D, 9)
    wq, wn = w[:_D], w[_D:]
    t9 = pl.pallas_call(
        _t9_body,
        grid=(_N // _QB,),
        in_specs=[
            pl.BlockSpec((_QB, _D), lambda i: (i, 0)),
            pl.BlockSpec((_QB, _D), lambda i: (i, 0)),
            pl.BlockSpec((_D, 9), lambda i: (0, 0)),
            pl.BlockSpec((_D, 9), lambda i: (0, 0)),
        ],
        out_specs=pl.BlockSpec((_QB, 9), lambda i: (i, 0)),
        out_shape=jax.ShapeDtypeStruct((_N, 9), jnp.float32),
    )(query, nmean, wq, wn)

    t3 = t9.T.reshape(9, 64, 64)
    out = pl.pallas_call(
        _shift_body,
        in_specs=[
            pl.BlockSpec((9, 64, 64), lambda: (0, 0, 0)),
            pl.BlockSpec((1, 1), lambda: (0, 0), memory_space=pltpu.SMEM),
        ],
        out_specs=pl.BlockSpec((64, 64), lambda: (0, 0)),
        out_shape=jax.ShapeDtypeStruct((64, 64), jnp.float32),
    )(t3, fcn_b.reshape(1, 1))
    return out.reshape(1, 1, 64, 64)


# speedup vs baseline: 1.5046x; 1.0358x over previous
import functools

import jax
import jax.numpy as jnp
from jax import lax
from jax.experimental import pallas as pl
from jax.experimental.pallas import tpu as pltpu
from jax.experimental.pallas import tpu_sc as plsc

_N = 4096
_D = 96
_M = 16384
_QB = 512
_C = 2048
_G = 32
_INF = 3.0e38
_BIGI = 2**30
_MAXI = 2**31 - 1

_NCORES = 2
_NSUB = 16
_NW = _NCORES * _NSUB
_QPW = _N // _NW


def _idx_body(q_ref, mb_ref, idx_ref, m2_ref):
    q = q_ref[...]
    nc = _M // _C
    ng = _C // _G

    @pl.when(pl.program_id(0) == 0)
    def _build_m2():
        for c in range(nc):
            mb = mb_ref[pl.ds(c * _C, _C), :]
            m2_ref[pl.ds(c * _C, _C), :] = jnp.sum(mb * mb, axis=1,
                                                   keepdims=True)

    qn = -2.0 * q
    ng = _C // _G
    pos5 = lax.broadcasted_iota(jnp.int32, (_C, _QB), 0) // ng
    gkeys = []
    for c in range(nc):
        mb = mb_ref[pl.ds(c * _C, _C), :]
        qm = lax.dot_general(mb, qn, (((1,), (1,)), ((), ())),
                             preferred_element_type=jnp.float32)
        s = m2_ref[pl.ds(c * _C, _C), :] + qm
        b = lax.bitcast_convert_type(s, jnp.int32)
        k = b ^ ((b >> 31) & jnp.int32(0x7FFFFFFF))
        k = (k & jnp.int32(~31)) | pos5
        h = _C // 2
        a1 = jnp.minimum(k[:h], k[h:])
        a2 = jnp.maximum(k[:h], k[h:])
        while h > ng:
            h //= 2
            b1, c1 = a1[:h], a1[h:]
            n1 = jnp.minimum(b1, c1)
            a2 = jnp.minimum(jnp.maximum(b1, c1),
                             jnp.minimum(a2[:h], a2[h:]))
            a1 = n1
        gkeys.append(jnp.concatenate([a1, a2], axis=0))
    ck = jnp.concatenate(gkeys, axis=0)
    ncand = 2 * ng * nc
    iota0 = lax.broadcasted_iota(jnp.int32, (ncand, _QB), 0)
    rows = []
    for _ in range(4):
        m = jnp.min(ck, axis=0, keepdims=True)
        hit = ck == m
        pos = jnp.min(jnp.where(hit, iota0, _BIGI), axis=0, keepdims=True)
        row = (pos // (2 * ng)) * _C + pos % ng + ng * (m & 31)
        rows.append(row)
        ck = jnp.where(iota0 == pos, _MAXI, ck)
    zero = jnp.zeros((4, _QB), jnp.int32)
    idx_ref[...] = jnp.concatenate(rows + [zero], axis=0)


@functools.cache
def _make_gather_mean():
    @functools.partial(
        pl.kernel,
        mesh=plsc.VectorSubcoreMesh(core_axis_name="c", subcore_axis_name="s"),
        out_type=jax.ShapeDtypeStruct((_N, _D), jnp.float32),
        scratch_types=[
            pltpu.VMEM((4, _QPW), jnp.int32),
            pltpu.VMEM((_QPW, 128), jnp.float32),
            pltpu.VMEM((_QPW, 128), jnp.float32),
            pltpu.VMEM((_QPW, 128), jnp.float32),
            pltpu.VMEM((_QPW, 128), jnp.float32),
            pltpu.VMEM((_QPW, _D), jnp.float32),
            pltpu.SemaphoreType.DMA,
        ],
    )
    def _gather_mean(idx_hbm, mb_hbm, out_hbm, idx_v,
                     r0, r1, r2, r3, out_v, sem):
        wid = lax.axis_index("s") * _NCORES + lax.axis_index("c")
        base = wid * _QPW
        pltpu.sync_copy(idx_hbm.at[pl.ds(0, 4), pl.ds(base, _QPW)], idx_v)
        rows = (r0, r1, r2, r3)
        copies = [pltpu.async_copy(mb_hbm.at[idx_v.at[k]], rows[k], sem)
                  for k in range(4)]
        for cp in copies:
            cp.wait()

        def body(qi, carry):
            for dk in range(_D // 16):
                sl = pl.ds(dk * 16, 16)
                out_v[qi, sl] = (r0[qi, sl] + r1[qi, sl]
                                 + r2[qi, sl] + r3[qi, sl]) * 0.25
            return carry

        lax.fori_loop(0, _QPW, body, 0)
        pltpu.sync_copy(out_v, out_hbm.at[pl.ds(base, _QPW)])

    return _gather_mean


def _t9_body(q_ref, n_ref, wq_ref, wn_ref, t_ref):
    t_ref[...] = (
        lax.dot_general(q_ref[...], wq_ref[...], (((1,), (0,)), ((), ())),
                        preferred_element_type=jnp.float32)
        + lax.dot_general(n_ref[...], wn_ref[...], (((1,), (0,)), ((), ())),
                          preferred_element_type=jnp.float32))


def _shift_body(t_ref, b_ref, out_ref):
    zrow = jnp.zeros((1, 64), jnp.float32)
    zcol = jnp.zeros((64, 1), jnp.float32)
    acc = jnp.zeros((64, 64), jnp.float32)
    for ky in range(3):
        for kx in range(3):
            p = t_ref[ky * 3 + kx]
            if ky == 0:
                p = jnp.concatenate([zrow, p[:63, :]], axis=0)
            elif ky == 2:
                p = jnp.concatenate([p[1:, :], zrow], axis=0)
            if kx == 0:
                p = jnp.concatenate([zcol, p[:, :63]], axis=1)
            elif kx == 2:
                p = jnp.concatenate([p[:, 1:], zcol], axis=1)
            acc = acc + p
    out_ref[...] = acc + b_ref[0, 0]


def kernel(query, memory_bank, fcn_w, fcn_b):
    idx_t = pl.pallas_call(
        _idx_body,
        grid=(_N // _QB,),
        in_specs=[
            pl.BlockSpec((_QB, _D), lambda i: (i, 0)),
            pl.BlockSpec((_M, _D), lambda i: (0, 0)),
        ],
        out_specs=pl.BlockSpec((8, _QB), lambda i: (0, i)),
        out_shape=jax.ShapeDtypeStruct((8, _N), jnp.int32),
        scratch_shapes=[pltpu.VMEM((_M, 1), jnp.float32)],
        compiler_params=pltpu.CompilerParams(
            dimension_semantics=("arbitrary",)),
    )(query, memory_bank)

    mb_pad = jnp.pad(memory_bank, ((0, 0), (0, 128 - _D)))
    nmean = _make_gather_mean()(idx_t, mb_pad)

    w = fcn_w[0].reshape(2 * _D, 9)
    wq, wn = w[:_D], w[_D:]
    t9 = pl.pallas_call(
        _t9_body,
        grid=(_N // _QB,),
        in_specs=[
            pl.BlockSpec((_QB, _D), lambda i: (i, 0)),
            pl.BlockSpec((_QB, _D), lambda i: (i, 0)),
            pl.BlockSpec((_D, 9), lambda i: (0, 0)),
            pl.BlockSpec((_D, 9), lambda i: (0, 0)),
        ],
        out_specs=pl.BlockSpec((_QB, 9), lambda i: (i, 0)),
        out_shape=jax.ShapeDtypeStruct((_N, 9), jnp.float32),
    )(query, nmean, wq, wn)

    t3 = t9.T.reshape(9, 64, 64)
    out = pl.pallas_call(
        _shift_body,
        in_specs=[
            pl.BlockSpec((9, 64, 64), lambda: (0, 0, 0)),
            pl.BlockSpec((1, 1), lambda: (0, 0), memory_space=pltpu.SMEM),
        ],
        out_specs=pl.BlockSpec((64, 64), lambda: (0, 0)),
        out_shape=jax.ShapeDtypeStruct((64, 64), jnp.float32),
    )(t3, fcn_b.reshape(1, 1))
    return out.reshape(1, 1, 64, 64)
